# Initial kernel scaffold; baseline (speedup 1.0000x reference)
import functools

import jax
import jax.numpy as jnp
from jax import lax
from jax.experimental import pallas as pl
from jax.experimental.pallas import tpu as pltpu
from jax.experimental.pallas import tpu_sc as plsc

N_NODES = 10000
N_EDGES = 320000
D = 128
NC = 2
NS = 16
NW = NC * NS
EPT = N_EDGES // NW
CH = 80
NCHUNK = EPT // CH
ROWS_PT = N_NODES // NS
ZROWS = 125


def _sc_aggregate(X, idxA, idxB, zeros):
    mesh = plsc.VectorSubcoreMesh(core_axis_name="c", subcore_axis_name="s")

    @functools.partial(
        pl.kernel,
        out_type=jax.ShapeDtypeStruct((NC, N_NODES, D), jnp.float32),
        mesh=mesh,
        scratch_types=[
            pltpu.VMEM_SHARED((N_NODES, D), jnp.float32),
            pltpu.VMEM((NCHUNK, CH), jnp.int32),
            pltpu.VMEM((NCHUNK, CH), jnp.int32),
            pltpu.VMEM((CH, D), jnp.float32),
            pltpu.VMEM((CH, D), jnp.float32),
            pltpu.VMEM((ZROWS, D), jnp.float32),
            pltpu.SemaphoreType.DMA,
            pltpu.SemaphoreType.DMA,
        ],
    )
    def agg_kernel(x_hbm, ia_hbm, ib_hbm, z_hbm, out_hbm,
                   acc_sh, ia_v, ib_v, rows0, rows1, zbuf, sem0, sem1):
        c = lax.axis_index("c")
        s = lax.axis_index("s")
        w = c * NS + s

        pltpu.sync_copy(z_hbm, zbuf)
        for r in range(ROWS_PT // ZROWS):
            pltpu.sync_copy(zbuf, acc_sh.at[pl.ds(s * ROWS_PT + r * ZROWS, ZROWS)])

        pltpu.sync_copy(ia_hbm.at[pl.ds(w * NCHUNK, NCHUNK)], ia_v)
        pltpu.sync_copy(ib_hbm.at[pl.ds(w * NCHUNK, NCHUNK)], ib_v)

        plsc.subcore_barrier()

        bufs = (rows0, rows1)
        sems = (sem0, sem1)

        pltpu.async_copy(x_hbm.at[ia_v.at[0]], rows0, sem0)
        pltpu.async_copy(x_hbm.at[ia_v.at[1]], rows1, sem1)

        @pl.loop(0, NCHUNK // 2)
        def _(j):
            for b in range(2):
                cix = j * 2 + b
                pltpu.make_async_copy(x_hbm.at[ia_v.at[cix]], bufs[b], sems[b]).wait()
                pltpu.sync_copy(bufs[b], acc_sh.at[ib_v.at[cix]], add=True)
                nxt = cix + 2

                @pl.when(nxt < NCHUNK)
                def _():
                    pltpu.async_copy(x_hbm.at[ia_v.at[nxt]], bufs[b], sems[b])

        last = NCHUNK - 1
        pltpu.make_async_copy(x_hbm.at[ia_v.at[last]], rows0, sem0).wait()
        pltpu.sync_copy(rows0, acc_sh.at[ib_v.at[last]], add=True)

        plsc.subcore_barrier()

        pltpu.sync_copy(acc_sh.at[pl.ds(s * ROWS_PT, ROWS_PT)],
                        out_hbm.at[c, pl.ds(s * ROWS_PT, ROWS_PT)])

    return agg_kernel(X, idxA, idxB, zeros)


def _mlp_body(x_ref, p_ref, w1_ref, b1_ref, w2_ref, b2_ref, o_ref):
    conv = x_ref[...] + p_ref[0] + p_ref[1]
    h = jnp.maximum(
        jnp.dot(conv, w1_ref[...], preferred_element_type=jnp.float32)
        + b1_ref[...], 0.0)
    o_ref[...] = jnp.maximum(
        jnp.dot(h, w2_ref[...], preferred_element_type=jnp.float32)
        + b2_ref[...], 0.0)


def _tc_mlp(X, partials, w1, b1, w2, b2):
    BR = 1000
    return pl.pallas_call(
        _mlp_body,
        grid=(N_NODES // BR,),
        in_specs=[
            pl.BlockSpec((BR, D), lambda i: (i, 0)),
            pl.BlockSpec((NC, BR, D), lambda i: (0, i, 0)),
            pl.BlockSpec((D, D), lambda i: (0, 0)),
            pl.BlockSpec((D,), lambda i: (0,)),
            pl.BlockSpec((D, D), lambda i: (0, 0)),
            pl.BlockSpec((D,), lambda i: (0,)),
        ],
        out_specs=pl.BlockSpec((BR, D), lambda i: (i, 0)),
        out_shape=jax.ShapeDtypeStruct((N_NODES, D), jnp.float32),
    )(X, partials, w1, b1, w2, b2)


def kernel(X, ref_A, ref_B, w1, b1, w2, b2):
    idxA = ref_A.astype(jnp.int32).reshape(N_EDGES // CH, CH)
    idxB = ref_B.astype(jnp.int32).reshape(N_EDGES // CH, CH)
    zeros = jnp.zeros((ZROWS, D), jnp.float32)
    partials = _sc_aggregate(X, idxA, idxB, zeros)
    return _tc_mlp(X, partials, w1, b1, w2, b2)


# speedup vs baseline: 12.6809x; 12.6809x over previous
import functools

import jax
import jax.numpy as jnp
from jax import lax
from jax.experimental import pallas as pl
from jax.experimental.pallas import tpu as pltpu
from jax.experimental.pallas import tpu_sc as plsc

N_NODES = 10000
N_EDGES = 320000
D = 128
NC = 2
NS = 16
NW = NC * NS
EPT = N_EDGES // NW
CH = 80
NCHUNK = EPT // CH
N_PAD = 10240
ROWS_PT = N_PAD // NS


def _sc_aggregate(X, idxA, idxB, zeros):
    mesh = plsc.VectorSubcoreMesh(core_axis_name="c", subcore_axis_name="s")

    @functools.partial(
        pl.kernel,
        out_type=jax.ShapeDtypeStruct((NC, N_PAD, D), jnp.float32),
        mesh=mesh,
        scratch_types=[
            pltpu.VMEM_SHARED((N_PAD, D), jnp.float32),
            pltpu.VMEM((NCHUNK, CH), jnp.int32),
            pltpu.VMEM((NCHUNK, CH), jnp.int32),
            pltpu.VMEM((CH, D), jnp.float32),
            pltpu.VMEM((CH, D), jnp.float32),
            pltpu.SemaphoreType.DMA,
            pltpu.SemaphoreType.DMA,
        ],
        compiler_params=pltpu.CompilerParams(use_tc_tiling_on_sc=False),
    )
    def agg_kernel(x_hbm, ia_hbm, ib_hbm, z_hbm, out_hbm,
                   acc_sh, ia_v, ib_v, rows0, rows1, sem0, sem1):
        c = lax.axis_index("c")
        s = lax.axis_index("s")
        w = c * NS + s

        pltpu.sync_copy(z_hbm, rows0)
        for r in range(ROWS_PT // CH):
            pltpu.sync_copy(rows0, acc_sh.at[pl.ds(s * ROWS_PT + r * CH, CH)])

        pltpu.sync_copy(ia_hbm.at[w], ia_v)
        pltpu.sync_copy(ib_hbm.at[w], ib_v)

        plsc.subcore_barrier()

        bufs = (rows0, rows1)
        sems = (sem0, sem1)

        pltpu.async_copy(x_hbm.at[ia_v.at[0]], rows0, sem0)
        pltpu.async_copy(x_hbm.at[ia_v.at[1]], rows1, sem1)

        @pl.loop(0, NCHUNK // 2)
        def _(j):
            for b in range(2):
                cix = j * 2 + b
                pltpu.make_async_copy(x_hbm.at[ia_v.at[cix]], bufs[b], sems[b]).wait()
                pltpu.sync_copy(bufs[b], acc_sh.at[ib_v.at[cix]], add=True)
                nxt = cix + 2

                @pl.when(nxt < NCHUNK)
                def _():
                    pltpu.async_copy(x_hbm.at[ia_v.at[nxt]], bufs[b], sems[b])

        if NCHUNK % 2 == 1:
            last = NCHUNK - 1
            pltpu.make_async_copy(x_hbm.at[ia_v.at[last]], rows0, sem0).wait()
            pltpu.sync_copy(rows0, acc_sh.at[ib_v.at[last]], add=True)

        plsc.subcore_barrier()

        pltpu.sync_copy(acc_sh.at[pl.ds(s * ROWS_PT, ROWS_PT)],
                        out_hbm.at[c, pl.ds(s * ROWS_PT, ROWS_PT)])

    return agg_kernel(X, idxA, idxB, zeros)


def _mlp_body(x_ref, p_ref, w1_ref, b1_ref, w2_ref, b2_ref, o_ref):
    conv = x_ref[...] + p_ref[0] + p_ref[1]
    h = jnp.maximum(
        jnp.dot(conv, w1_ref[...], preferred_element_type=jnp.float32)
        + b1_ref[...], 0.0)
    o_ref[...] = jnp.maximum(
        jnp.dot(h, w2_ref[...], preferred_element_type=jnp.float32)
        + b2_ref[...], 0.0)


def _tc_mlp(X, partials, w1, b1, w2, b2):
    BR = 1000
    return pl.pallas_call(
        _mlp_body,
        grid=(N_NODES // BR,),
        in_specs=[
            pl.BlockSpec((BR, D), lambda i: (i, 0)),
            pl.BlockSpec((NC, BR, D), lambda i: (0, i, 0)),
            pl.BlockSpec((D, D), lambda i: (0, 0)),
            pl.BlockSpec((D,), lambda i: (0,)),
            pl.BlockSpec((D, D), lambda i: (0, 0)),
            pl.BlockSpec((D,), lambda i: (0,)),
        ],
        out_specs=pl.BlockSpec((BR, D), lambda i: (i, 0)),
        out_shape=jax.ShapeDtypeStruct((N_NODES, D), jnp.float32),
    )(X, partials, w1, b1, w2, b2)


def kernel(X, ref_A, ref_B, w1, b1, w2, b2):
    idxA = ref_A.astype(jnp.int32).reshape(NW, NCHUNK, CH)
    idxB = ref_B.astype(jnp.int32).reshape(NW, NCHUNK, CH)
    zeros = jnp.zeros((CH, D), jnp.float32)
    partials = _sc_aggregate(X, idxA, idxB, zeros)
    return _tc_mlp(X, partials, w1, b1, w2, b2)


# speedup vs baseline: 13.3296x; 1.0512x over previous
import functools

import jax
import jax.numpy as jnp
from jax import lax
from jax.experimental import pallas as pl
from jax.experimental.pallas import tpu as pltpu
from jax.experimental.pallas import tpu_sc as plsc

N_NODES = 10000
N_EDGES = 320000
D = 128
NC = 2
NS = 16
NW = NC * NS
EPT = N_EDGES // NW
CH = 80
NCHUNK = EPT // CH
N_PAD = 10240
ROWS_PT = N_PAD // NS


def _sc_aggregate(X, idxA, idxB, zeros):
    mesh = plsc.VectorSubcoreMesh(core_axis_name="c", subcore_axis_name="s")

    @functools.partial(
        pl.kernel,
        out_type=jax.ShapeDtypeStruct((NC, N_PAD, D), jnp.bfloat16),
        mesh=mesh,
        scratch_types=[
            pltpu.VMEM_SHARED((N_PAD, D), jnp.bfloat16),
            pltpu.VMEM((NCHUNK, CH), jnp.int32),
            pltpu.VMEM((NCHUNK, CH), jnp.int32),
            pltpu.VMEM((CH, D), jnp.bfloat16),
            pltpu.VMEM((CH, D), jnp.bfloat16),
            pltpu.SemaphoreType.DMA,
            pltpu.SemaphoreType.DMA,
        ],
        compiler_params=pltpu.CompilerParams(use_tc_tiling_on_sc=False),
    )
    def agg_kernel(x_hbm, ia_hbm, ib_hbm, z_hbm, out_hbm,
                   acc_sh, ia_v, ib_v, rows0, rows1, sem0, sem1):
        c = lax.axis_index("c")
        s = lax.axis_index("s")
        w = c * NS + s

        pltpu.sync_copy(z_hbm, rows0)
        for r in range(ROWS_PT // CH):
            pltpu.sync_copy(rows0, acc_sh.at[pl.ds(s * ROWS_PT + r * CH, CH)])

        pltpu.sync_copy(ia_hbm.at[w], ia_v)
        pltpu.sync_copy(ib_hbm.at[w], ib_v)

        plsc.subcore_barrier()

        bufs = (rows0, rows1)
        sems = (sem0, sem1)

        pltpu.async_copy(x_hbm.at[ia_v.at[0]], rows0, sem0)
        pltpu.async_copy(x_hbm.at[ia_v.at[1]], rows1, sem1)

        @pl.loop(0, NCHUNK // 2)
        def _(j):
            for b in range(2):
                cix = j * 2 + b
                pltpu.make_async_copy(x_hbm.at[ia_v.at[cix]], bufs[b], sems[b]).wait()
                pltpu.sync_copy(bufs[b], acc_sh.at[ib_v.at[cix]], add=True)
                nxt = cix + 2

                @pl.when(nxt < NCHUNK)
                def _():
                    pltpu.async_copy(x_hbm.at[ia_v.at[nxt]], bufs[b], sems[b])

        if NCHUNK % 2 == 1:
            last = NCHUNK - 1
            pltpu.make_async_copy(x_hbm.at[ia_v.at[last]], rows0, sem0).wait()
            pltpu.sync_copy(rows0, acc_sh.at[ib_v.at[last]], add=True)

        plsc.subcore_barrier()

        pltpu.sync_copy(acc_sh.at[pl.ds(s * ROWS_PT, ROWS_PT)],
                        out_hbm.at[c, pl.ds(s * ROWS_PT, ROWS_PT)])

    return agg_kernel(X, idxA, idxB, zeros)


def _mlp_body(x_ref, p_ref, w1_ref, b1_ref, w2_ref, b2_ref, o_ref):
    conv = x_ref[...] + (p_ref[0] + p_ref[1]).astype(jnp.float32)
    h = jnp.maximum(
        jnp.dot(conv, w1_ref[...], preferred_element_type=jnp.float32)
        + b1_ref[...], 0.0)
    o_ref[...] = jnp.maximum(
        jnp.dot(h, w2_ref[...], preferred_element_type=jnp.float32)
        + b2_ref[...], 0.0)


def _tc_mlp(X, partials, w1, b1, w2, b2):
    BR = 1000
    return pl.pallas_call(
        _mlp_body,
        grid=(N_NODES // BR,),
        in_specs=[
            pl.BlockSpec((BR, D), lambda i: (i, 0)),
            pl.BlockSpec((NC, BR, D), lambda i: (0, i, 0)),
            pl.BlockSpec((D, D), lambda i: (0, 0)),
            pl.BlockSpec((D,), lambda i: (0,)),
            pl.BlockSpec((D, D), lambda i: (0, 0)),
            pl.BlockSpec((D,), lambda i: (0,)),
        ],
        out_specs=pl.BlockSpec((BR, D), lambda i: (i, 0)),
        out_shape=jax.ShapeDtypeStruct((N_NODES, D), jnp.float32),
    )(X, partials, w1, b1, w2, b2)


def kernel(X, ref_A, ref_B, w1, b1, w2, b2):
    idxA = ref_A.astype(jnp.int32).reshape(NW, NCHUNK, CH)
    idxB = ref_B.astype(jnp.int32).reshape(NW, NCHUNK, CH)
    zeros = jnp.zeros((CH, D), jnp.bfloat16)
    partials = _sc_aggregate(X.astype(jnp.bfloat16), idxA, idxB, zeros)
    return _tc_mlp(X, partials, w1, b1, w2, b2)
